# baseline (speedup 1.0000x reference)
import jax
import jax.numpy as jnp
from jax import lax
from jax.experimental import pallas as pl
from jax.experimental.pallas import tpu as pltpu
from jax.experimental.pallas import tpu_sc as plsc

N = 10000
NPAD = 10240
E = 320000
D = 128

NC = 2
NS = 16
NW = NC * NS

EPW = E // NW
CHUNK = 96
NFULL = EPW // CHUNK
TAIL = EPW - NFULL * CHUNK
RPT = NPAD // NS
RB = 80
NRB = RPT // RB

HCHUNK = 80
HNCHUNK = EPW // HCHUNK

HIST_W = 16

BR = 2000

_mesh = plsc.VectorSubcoreMesh(core_axis_name="c", subcore_axis_name="s")


def _hist_body(dst_hbm, out_hbm, idx_v, ones_v, zero_v, hist_sh, sem):
  cid = lax.axis_index("c")
  sid = lax.axis_index("s")
  wid = cid * NS + sid

  @pl.loop(0, HCHUNK)
  def _(r):
    ones_v[r, :] = jnp.ones((HIST_W,), jnp.float32)

  @pl.loop(0, RB)
  def _(r):
    zero_v[r, :] = jnp.zeros((HIST_W,), jnp.float32)

  @pl.loop(0, NRB)
  def _(k):
    pltpu.sync_copy(zero_v, hist_sh.at[pl.ds(sid * RPT + k * RB, RB)])

  pltpu.sync_copy(dst_hbm.at[wid], idx_v)
  plsc.subcore_barrier()

  @pl.loop(0, HNCHUNK)
  def _(j):
    pltpu.sync_copy(ones_v, hist_sh.at[idx_v.at[j]], add=True)

  plsc.subcore_barrier()

  @pl.loop(0, NRB)
  def _(k):
    sl = pl.ds(sid * RPT + k * RB, RB)
    pltpu.sync_copy(hist_sh.at[sl], out_hbm.at[cid, sl])


def _sc_histogram(dst_idx):
  return pl.kernel(
      _hist_body,
      out_type=jax.ShapeDtypeStruct((NC, NPAD, HIST_W), jnp.float32),
      mesh=_mesh,
      scratch_types=[
          pltpu.VMEM((HNCHUNK, HCHUNK), jnp.int32),
          pltpu.VMEM((HCHUNK, HIST_W), jnp.float32),
          pltpu.VMEM((RB, HIST_W), jnp.float32),
          pltpu.VMEM_SHARED((NPAD, HIST_W), jnp.float32),
          pltpu.SemaphoreType.DMA,
      ],
  )(dst_idx)


def _scat_body(h_hbm, src_hbm, dst_hbm, out_hbm, src_v, dst_v, rows_v,
               acc_sh, gsem0, gsem1, ssem0, ssem1):
  cid = lax.axis_index("c")
  sid = lax.axis_index("s")
  wid = cid * NS + sid

  @pl.loop(0, RB)
  def _(r):
    @pl.loop(0, D, step=16)
    def _(c):
      rows_v[0, r, pl.ds(c, 16)] = jnp.zeros((16,), jnp.float32)

  @pl.loop(0, NRB)
  def _(k):
    pltpu.sync_copy(rows_v.at[0, pl.ds(0, RB)],
                    acc_sh.at[pl.ds(sid * RPT + k * RB, RB)])

  pltpu.sync_copy(src_hbm.at[wid, 0], src_v)
  pltpu.sync_copy(dst_hbm.at[wid, 0], dst_v)
  plsc.subcore_barrier()

  def gather_start(j, p, sem):
    pltpu.async_copy(
        h_hbm.at[src_v.at[pl.ds(j * CHUNK, CHUNK)]], rows_v.at[p], sem)

  def gather_wait(p, sem):
    pltpu.make_async_copy(
        h_hbm.at[src_v.at[pl.ds(0, CHUNK)]], rows_v.at[p], sem).wait()

  def scatter_add(j, p):
    pltpu.sync_copy(
        rows_v.at[p], acc_sh.at[dst_v.at[pl.ds(j * CHUNK, CHUNK)]], add=True)

  gather_start(0, 0, gsem0)

  @pl.loop(0, NFULL, step=2)
  def _(j):
    gather_wait(0, gsem0)
    gather_start(j + 1, 1, gsem1)
    scatter_add(j, 0)
    gather_wait(1, gsem1)

    @pl.when(j + 2 < NFULL)
    def _():
      gather_start(j + 2, 0, gsem0)

    scatter_add(j + 1, 1)

  pltpu.async_copy(
      h_hbm.at[src_v.at[pl.ds(NFULL * CHUNK, TAIL)]],
      rows_v.at[0, pl.ds(0, TAIL)], gsem0)
  pltpu.make_async_copy(
      h_hbm.at[src_v.at[pl.ds(0, TAIL)]],
      rows_v.at[0, pl.ds(0, TAIL)], gsem0).wait()
  pltpu.sync_copy(
      rows_v.at[0, pl.ds(0, TAIL)],
      acc_sh.at[dst_v.at[pl.ds(NFULL * CHUNK, TAIL)]], add=True)

  plsc.subcore_barrier()

  @pl.loop(0, NRB)
  def _(k):
    sl = pl.ds(sid * RPT + k * RB, RB)
    pltpu.sync_copy(acc_sh.at[sl], out_hbm.at[cid, sl])


def _sc_scatter(h, src_idx, dst_idx):
  return pl.kernel(
      _scat_body,
      out_type=jax.ShapeDtypeStruct((NC, NPAD, D), jnp.float32),
      mesh=_mesh,
      scratch_types=[
          pltpu.VMEM((EPW,), jnp.int32),
          pltpu.VMEM((EPW,), jnp.int32),
          pltpu.VMEM((2, CHUNK, D), jnp.float32),
          pltpu.VMEM_SHARED((NPAD, D), jnp.float32),
          pltpu.SemaphoreType.DMA,
          pltpu.SemaphoreType.DMA,
          pltpu.SemaphoreType.DMA,
          pltpu.SemaphoreType.DMA,
      ],
  )(h, src_idx, dst_idx)


def _dinv_from_hist(hist_blk):
  deg = jnp.sum(hist_blk, axis=(0, 2)) + 1.0
  return lax.rsqrt(deg)[:, None]


def _tc_first_body(hist_ref, x_ref, w_ref, out_ref):
  dinv = _dinv_from_hist(hist_ref[...])
  h = jnp.dot(x_ref[...], w_ref[...], preferred_element_type=jnp.float32)
  out_ref[...] = h * dinv


def _tc_first(hist, x, W):
  return pl.pallas_call(
      _tc_first_body,
      grid=(N // BR,),
      in_specs=[
          pl.BlockSpec((NC, BR, HIST_W), lambda i: (0, i, 0)),
          pl.BlockSpec((BR, D), lambda i: (i, 0)),
          pl.BlockSpec((D, D), lambda i: (0, 0)),
      ],
      out_specs=pl.BlockSpec((BR, D), lambda i: (i, 0)),
      out_shape=jax.ShapeDtypeStruct((N, D), jnp.float32),
  )(hist, x, W)


def _tc_advance_body(hist_ref, acc_ref, h_ref, b_ref, w_ref, out_ref):
  dinv = _dinv_from_hist(hist_ref[...])
  z = (acc_ref[0] + acc_ref[1] + h_ref[...]) * dinv + b_ref[...]
  xn = jnp.maximum(z, 0.0)
  out_ref[...] = jnp.dot(
      xn, w_ref[...], preferred_element_type=jnp.float32) * dinv


def _tc_advance(hist, acc, h, b, W):
  return pl.pallas_call(
      _tc_advance_body,
      grid=(N // BR,),
      in_specs=[
          pl.BlockSpec((NC, BR, HIST_W), lambda i: (0, i, 0)),
          pl.BlockSpec((NC, BR, D), lambda i: (0, i, 0)),
          pl.BlockSpec((BR, D), lambda i: (i, 0)),
          pl.BlockSpec((1, D), lambda i: (0, 0)),
          pl.BlockSpec((D, D), lambda i: (0, 0)),
      ],
      out_specs=pl.BlockSpec((BR, D), lambda i: (i, 0)),
      out_shape=jax.ShapeDtypeStruct((N, D), jnp.float32),
  )(hist, acc, h, b, W)


def _tc_final_body(hist_ref, acc_ref, h_ref, b_ref, out_ref):
  dinv = _dinv_from_hist(hist_ref[...])
  out_ref[...] = (acc_ref[0] + acc_ref[1] + h_ref[...]) * dinv + b_ref[...]


def _tc_final(hist, acc, h, b):
  return pl.pallas_call(
      _tc_final_body,
      grid=(N // BR,),
      in_specs=[
          pl.BlockSpec((NC, BR, HIST_W), lambda i: (0, i, 0)),
          pl.BlockSpec((NC, BR, D), lambda i: (0, i, 0)),
          pl.BlockSpec((BR, D), lambda i: (i, 0)),
          pl.BlockSpec((1, D), lambda i: (0, 0)),
      ],
      out_specs=pl.BlockSpec((BR, D), lambda i: (i, 0)),
      out_shape=jax.ShapeDtypeStruct((N, D), jnp.float32),
  )(hist, acc, h, b)


def kernel(x, edge_index, W1, b1, W2, b2, W3, b3):
  src = edge_index[0].reshape(NW, 1, EPW)
  dst = edge_index[1].reshape(NW, 1, EPW)
  dst_h = edge_index[1].reshape(NW, HNCHUNK, HCHUNK)
  b1r = b1.reshape(1, D)
  b2r = b2.reshape(1, D)
  b3r = b3.reshape(1, D)

  hist = _sc_histogram(dst_h)
  h1 = _tc_first(hist, x, W1)
  acc1 = _sc_scatter(h1, src, dst)
  h2 = _tc_advance(hist, acc1, h1, b1r, W2)
  acc2 = _sc_scatter(h2, src, dst)
  h3 = _tc_advance(hist, acc2, h2, b2r, W3)
  acc3 = _sc_scatter(h3, src, dst)
  out = _tc_final(hist, acc3, h3, b3r)
  return out


# speedup vs baseline: 1.0599x; 1.0599x over previous
import jax
import jax.numpy as jnp
from jax import lax
from jax.experimental import pallas as pl
from jax.experimental.pallas import tpu as pltpu
from jax.experimental.pallas import tpu_sc as plsc

N = 10000
NPAD = 10240
E = 320000
D = 128

NC = 2
NS = 16
NW = NC * NS

EPW = E // NW
CHUNK = 112
NFULL = EPW // CHUNK
TAIL = EPW - NFULL * CHUNK
RPT = NPAD // NS
RB = 80
NRB = RPT // RB

HCHUNK = 125
HNCHUNK = EPW // HCHUNK

HIST_W = 16

BR = 2000

_mesh = plsc.VectorSubcoreMesh(core_axis_name="c", subcore_axis_name="s")


def _hist_body(dst_hbm, out_hbm, idx_v, ones_v, zero_v, hist_sh, sem):
  cid = lax.axis_index("c")
  sid = lax.axis_index("s")
  wid = cid * NS + sid

  @pl.loop(0, HCHUNK)
  def _(r):
    ones_v[r, :] = jnp.ones((HIST_W,), jnp.float32)

  @pl.loop(0, RB)
  def _(r):
    zero_v[r, :] = jnp.zeros((HIST_W,), jnp.float32)

  @pl.loop(0, NRB)
  def _(k):
    pltpu.sync_copy(zero_v, hist_sh.at[pl.ds(sid * RPT + k * RB, RB)])

  pltpu.sync_copy(dst_hbm.at[wid], idx_v)
  plsc.subcore_barrier()

  @pl.loop(0, HNCHUNK)
  def _(j):
    pltpu.sync_copy(ones_v, hist_sh.at[idx_v.at[j]], add=True)

  plsc.subcore_barrier()

  @pl.loop(0, NRB)
  def _(k):
    sl = pl.ds(sid * RPT + k * RB, RB)
    pltpu.sync_copy(hist_sh.at[sl], out_hbm.at[cid, sl])


def _sc_histogram(dst_idx):
  return pl.kernel(
      _hist_body,
      out_type=jax.ShapeDtypeStruct((NC, NPAD, HIST_W), jnp.float32),
      mesh=_mesh,
      scratch_types=[
          pltpu.VMEM((HNCHUNK, HCHUNK), jnp.int32),
          pltpu.VMEM((HCHUNK, HIST_W), jnp.float32),
          pltpu.VMEM((RB, HIST_W), jnp.float32),
          pltpu.VMEM_SHARED((NPAD, HIST_W), jnp.float32),
          pltpu.SemaphoreType.DMA,
      ],
  )(dst_idx)


def _scat_body(h_hbm, src_hbm, dst_hbm, out_hbm, src_v, dst_v, rows_v,
               acc_sh, gsem0, gsem1, ssem0, ssem1):
  cid = lax.axis_index("c")
  sid = lax.axis_index("s")
  wid = cid * NS + sid

  @pl.loop(0, RB)
  def _(r):
    @pl.loop(0, D, step=16)
    def _(c):
      rows_v[0, r, pl.ds(c, 16)] = jnp.zeros((16,), jnp.float32)

  @pl.loop(0, NRB)
  def _(k):
    pltpu.sync_copy(rows_v.at[0, pl.ds(0, RB)],
                    acc_sh.at[pl.ds(sid * RPT + k * RB, RB)])

  pltpu.sync_copy(src_hbm.at[wid, 0], src_v)
  pltpu.sync_copy(dst_hbm.at[wid, 0], dst_v)
  plsc.subcore_barrier()

  def gather_start(j, p, sem):
    pltpu.async_copy(
        h_hbm.at[src_v.at[pl.ds(j * CHUNK, CHUNK)]], rows_v.at[p], sem)

  def gather_wait(p, sem):
    pltpu.make_async_copy(
        h_hbm.at[src_v.at[pl.ds(0, CHUNK)]], rows_v.at[p], sem).wait()

  def scatter_add(j, p):
    pltpu.sync_copy(
        rows_v.at[p], acc_sh.at[dst_v.at[pl.ds(j * CHUNK, CHUNK)]], add=True)

  gather_start(0, 0, gsem0)

  @pl.loop(0, NFULL - 1, step=2)
  def _(j):
    gather_wait(0, gsem0)
    gather_start(j + 1, 1, gsem1)
    scatter_add(j, 0)
    gather_wait(1, gsem1)
    gather_start(j + 2, 0, gsem0)
    scatter_add(j + 1, 1)

  gather_wait(0, gsem0)
  pltpu.async_copy(
      h_hbm.at[src_v.at[pl.ds(NFULL * CHUNK, TAIL)]],
      rows_v.at[1, pl.ds(0, TAIL)], gsem1)
  scatter_add(NFULL - 1, 0)
  pltpu.make_async_copy(
      h_hbm.at[src_v.at[pl.ds(0, TAIL)]],
      rows_v.at[1, pl.ds(0, TAIL)], gsem1).wait()
  pltpu.sync_copy(
      rows_v.at[1, pl.ds(0, TAIL)],
      acc_sh.at[dst_v.at[pl.ds(NFULL * CHUNK, TAIL)]], add=True)

  plsc.subcore_barrier()

  @pl.loop(0, NRB)
  def _(k):
    sl = pl.ds(sid * RPT + k * RB, RB)
    pltpu.sync_copy(acc_sh.at[sl], out_hbm.at[cid, sl])


def _sc_scatter(h, src_idx, dst_idx):
  return pl.kernel(
      _scat_body,
      out_type=jax.ShapeDtypeStruct((NC, NPAD, D), jnp.float32),
      mesh=_mesh,
      scratch_types=[
          pltpu.VMEM((EPW,), jnp.int32),
          pltpu.VMEM((EPW,), jnp.int32),
          pltpu.VMEM((2, CHUNK, D), jnp.float32),
          pltpu.VMEM_SHARED((NPAD, D), jnp.float32),
          pltpu.SemaphoreType.DMA,
          pltpu.SemaphoreType.DMA,
          pltpu.SemaphoreType.DMA,
          pltpu.SemaphoreType.DMA,
      ],
  )(h, src_idx, dst_idx)


def _dinv_from_hist(hist_blk):
  deg = jnp.sum(hist_blk, axis=(0, 2)) + 1.0
  return lax.rsqrt(deg)[:, None]


def _tc_first_body(hist_ref, x_ref, w_ref, out_ref):
  dinv = _dinv_from_hist(hist_ref[...])
  h = jnp.dot(x_ref[...], w_ref[...], preferred_element_type=jnp.float32)
  out_ref[...] = h * dinv


def _tc_first(hist, x, W):
  return pl.pallas_call(
      _tc_first_body,
      grid=(N // BR,),
      in_specs=[
          pl.BlockSpec((NC, BR, HIST_W), lambda i: (0, i, 0)),
          pl.BlockSpec((BR, D), lambda i: (i, 0)),
          pl.BlockSpec((D, D), lambda i: (0, 0)),
      ],
      out_specs=pl.BlockSpec((BR, D), lambda i: (i, 0)),
      out_shape=jax.ShapeDtypeStruct((N, D), jnp.float32),
  )(hist, x, W)


def _tc_advance_body(hist_ref, acc_ref, h_ref, b_ref, w_ref, out_ref):
  dinv = _dinv_from_hist(hist_ref[...])
  z = (acc_ref[0] + acc_ref[1] + h_ref[...]) * dinv + b_ref[...]
  xn = jnp.maximum(z, 0.0)
  out_ref[...] = jnp.dot(
      xn, w_ref[...], preferred_element_type=jnp.float32) * dinv


def _tc_advance(hist, acc, h, b, W):
  return pl.pallas_call(
      _tc_advance_body,
      grid=(N // BR,),
      in_specs=[
          pl.BlockSpec((NC, BR, HIST_W), lambda i: (0, i, 0)),
          pl.BlockSpec((NC, BR, D), lambda i: (0, i, 0)),
          pl.BlockSpec((BR, D), lambda i: (i, 0)),
          pl.BlockSpec((1, D), lambda i: (0, 0)),
          pl.BlockSpec((D, D), lambda i: (0, 0)),
      ],
      out_specs=pl.BlockSpec((BR, D), lambda i: (i, 0)),
      out_shape=jax.ShapeDtypeStruct((N, D), jnp.float32),
  )(hist, acc, h, b, W)


def _tc_final_body(hist_ref, acc_ref, h_ref, b_ref, out_ref):
  dinv = _dinv_from_hist(hist_ref[...])
  out_ref[...] = (acc_ref[0] + acc_ref[1] + h_ref[...]) * dinv + b_ref[...]


def _tc_final(hist, acc, h, b):
  return pl.pallas_call(
      _tc_final_body,
      grid=(N // BR,),
      in_specs=[
          pl.BlockSpec((NC, BR, HIST_W), lambda i: (0, i, 0)),
          pl.BlockSpec((NC, BR, D), lambda i: (0, i, 0)),
          pl.BlockSpec((BR, D), lambda i: (i, 0)),
          pl.BlockSpec((1, D), lambda i: (0, 0)),
      ],
      out_specs=pl.BlockSpec((BR, D), lambda i: (i, 0)),
      out_shape=jax.ShapeDtypeStruct((N, D), jnp.float32),
  )(hist, acc, h, b)


def kernel(x, edge_index, W1, b1, W2, b2, W3, b3):
  src = edge_index[0].reshape(NW, 1, EPW)
  dst = edge_index[1].reshape(NW, 1, EPW)
  dst_h = edge_index[1].reshape(NW, HNCHUNK, HCHUNK)
  b1r = b1.reshape(1, D)
  b2r = b2.reshape(1, D)
  b3r = b3.reshape(1, D)

  hist = _sc_histogram(dst_h)
  h1 = _tc_first(hist, x, W1)
  acc1 = _sc_scatter(h1, src, dst)
  h2 = _tc_advance(hist, acc1, h1, b1r, W2)
  acc2 = _sc_scatter(h2, src, dst)
  h3 = _tc_advance(hist, acc2, h2, b2r, W3)
  acc3 = _sc_scatter(h3, src, dst)
  out = _tc_final(hist, acc3, h3, b3r)
  return out
